# Initial kernel scaffold; baseline (speedup 1.0000x reference)
import jax
import jax.numpy as jnp
from jax.experimental import pallas as pl


def kernel(x, gemm_edges, W, b):
    raise NotImplementedError("write your pallas kernel here")


# speedup vs baseline: 5.5044x; 5.5044x over previous
import jax
import jax.numpy as jnp
from jax import lax
from jax.experimental import pallas as pl
from jax.experimental.pallas import tpu as pltpu
from jax.experimental.pallas import tpu_sc as plsc

B, C, E, O, K = 1, 128, 160000, 128, 5

NC, NS = 2, 16
NW = NC * NS
CH = 128
NCH = E // CH
ITERS = (NCH + NW - 1) // NW

BE = 1280


def _sc_gather_body(xT_ref, idx_ref, g_ref, idx_v, rows_v, sem):
    cid = lax.axis_index("c")
    sid = lax.axis_index("s")
    wid = sid * NC + cid

    def chunk_body(i, carry):
        c = wid + i * NW

        @pl.when(c < NCH)
        def _():
            pltpu.sync_copy(idx_ref.at[c], idx_v)
            cps = [
                pltpu.async_copy(xT_ref.at[idx_v.at[j]], rows_v.at[j], sem)
                for j in range(4)
            ]
            for cp in cps:
                cp.wait()
            for j in range(4):
                pltpu.sync_copy(rows_v.at[j], g_ref.at[j, pl.ds(c * CH, CH), :])

        return carry

    lax.fori_loop(0, ITERS, chunk_body, 0)


_gather = pl.kernel(
    _sc_gather_body,
    out_type=jax.ShapeDtypeStruct((4, E, C), jnp.float32),
    mesh=plsc.VectorSubcoreMesh(
        core_axis_name="c", subcore_axis_name="s", num_cores=NC, num_subcores=NS
    ),
    scratch_types=[
        pltpu.VMEM((4, CH), jnp.int32),
        pltpu.VMEM((4, CH, C), jnp.float32),
        pltpu.SemaphoreType.DMA,
    ],
)


def _tc_conv_body(x_ref, g_ref, w_ref, b_ref, o_ref):
    r1 = g_ref[0]
    r2 = g_ref[1]
    r3 = g_ref[2]
    r4 = g_ref[3]
    dn = (((1,), (1,)), ((), ()))
    a = jnp.dot(w_ref[0], x_ref[...], preferred_element_type=jnp.float32)
    a += lax.dot_general(w_ref[1], r1 + r3, dn, preferred_element_type=jnp.float32)
    a += lax.dot_general(w_ref[2], r2 + r4, dn, preferred_element_type=jnp.float32)
    a += lax.dot_general(w_ref[3], jnp.abs(r1 - r3), dn, preferred_element_type=jnp.float32)
    a += lax.dot_general(w_ref[4], jnp.abs(r2 - r4), dn, preferred_element_type=jnp.float32)
    o_ref[...] = a + b_ref[...]


_conv = pl.pallas_call(
    _tc_conv_body,
    grid=(E // BE,),
    in_specs=[
        pl.BlockSpec((C, BE), lambda i: (0, i)),
        pl.BlockSpec((4, BE, C), lambda i: (0, i, 0)),
        pl.BlockSpec((K, O, C), lambda i: (0, 0, 0)),
        pl.BlockSpec((O, 1), lambda i: (0, 0)),
    ],
    out_specs=pl.BlockSpec((O, BE), lambda i: (0, i)),
    out_shape=jax.ShapeDtypeStruct((O, E), jnp.float32),
)


def kernel(x, gemm_edges, W, b):
    x2 = x[0]
    xT = x2.T
    idx_arr = gemm_edges[0].reshape(NCH, CH, 4).transpose(0, 2, 1)
    Wst = jnp.transpose(W[:, :, 0, :], (2, 0, 1))
    b2 = b.reshape(O, 1)
    G = _gather(xT, idx_arr)
    out2 = _conv(x2, G, Wst, b2)
    return out2[None, :, :, None]


# speedup vs baseline: 5.8953x; 1.0710x over previous
import jax
import jax.numpy as jnp
from jax import lax
from jax.experimental import pallas as pl
from jax.experimental.pallas import tpu as pltpu
from jax.experimental.pallas import tpu_sc as plsc

B, C, E, O, K = 1, 128, 160000, 128, 5

NC, NS = 2, 16
NW = NC * NS
CH = 128
NCH = E // CH
ITERS = (NCH + NW - 1) // NW

BE = 1280


def _sc_gather_body(xT_ref, idx_ref, g_ref, idx_v, rows_v,
                    gs0, gs1, gs2, gs3, ws0, ws1, ws2, ws3, isem):
    gsem = (gs0, gs1, gs2, gs3)
    wsem = (ws0, ws1, ws2, ws3)
    cid = lax.axis_index("c")
    sid = lax.axis_index("s")
    wid = sid * NC + cid

    pltpu.async_copy(idx_ref.at[wid], idx_v.at[0], isem)

    def chunk_body(i, carry):
        c = wid + i * NW
        p = lax.rem(i, 2)

        @pl.when(c < NCH)
        def _():
            pltpu.make_async_copy(idx_ref.at[c], idx_v.at[p], isem).wait()

            @pl.when(c + NW < NCH)
            def _():
                pltpu.async_copy(idx_ref.at[c + NW], idx_v.at[1 - p], isem)

            for j in range(4):
                @pl.when(i >= 1)
                def _():
                    pltpu.make_async_copy(
                        rows_v.at[j], g_ref.at[j, pl.ds(0, CH), :], wsem[j]
                    ).wait()

                pltpu.async_copy(
                    xT_ref.at[idx_v.at[p, j]], rows_v.at[j], gsem[j]
                )
            for j in range(4):
                pltpu.make_async_copy(
                    xT_ref.at[idx_v.at[p, j]], rows_v.at[j], gsem[j]
                ).wait()
                pltpu.async_copy(
                    rows_v.at[j], g_ref.at[j, pl.ds(c * CH, CH), :], wsem[j]
                )

        return carry

    lax.fori_loop(0, ITERS, chunk_body, 0)
    for j in range(4):
        pltpu.make_async_copy(
            rows_v.at[j], g_ref.at[j, pl.ds(0, CH), :], wsem[j]
        ).wait()


_gather = pl.kernel(
    _sc_gather_body,
    out_type=jax.ShapeDtypeStruct((4, E, C), jnp.float32),
    mesh=plsc.VectorSubcoreMesh(
        core_axis_name="c", subcore_axis_name="s", num_cores=NC, num_subcores=NS
    ),
    scratch_types=[
        pltpu.VMEM((2, 4, CH), jnp.int32),
        pltpu.VMEM((4, CH, C), jnp.float32),
    ] + [pltpu.SemaphoreType.DMA] * 9,
)


def _tc_conv_body(x_ref, g_ref, w_ref, b_ref, o_ref):
    r1 = g_ref[0]
    r2 = g_ref[1]
    r3 = g_ref[2]
    r4 = g_ref[3]
    dn = (((1,), (1,)), ((), ()))
    a = jnp.dot(w_ref[0], x_ref[...], preferred_element_type=jnp.float32)
    a += lax.dot_general(w_ref[1], r1 + r3, dn, preferred_element_type=jnp.float32)
    a += lax.dot_general(w_ref[2], r2 + r4, dn, preferred_element_type=jnp.float32)
    a += lax.dot_general(w_ref[3], jnp.abs(r1 - r3), dn, preferred_element_type=jnp.float32)
    a += lax.dot_general(w_ref[4], jnp.abs(r2 - r4), dn, preferred_element_type=jnp.float32)
    o_ref[...] = a + b_ref[...]


_conv = pl.pallas_call(
    _tc_conv_body,
    grid=(E // BE,),
    in_specs=[
        pl.BlockSpec((C, BE), lambda i: (0, i)),
        pl.BlockSpec((4, BE, C), lambda i: (0, i, 0)),
        pl.BlockSpec((K, O, C), lambda i: (0, 0, 0)),
        pl.BlockSpec((O, 1), lambda i: (0, 0)),
    ],
    out_specs=pl.BlockSpec((O, BE), lambda i: (0, i)),
    out_shape=jax.ShapeDtypeStruct((O, E), jnp.float32),
)


def kernel(x, gemm_edges, W, b):
    x2 = x[0]
    xT = x2.T
    idx_arr = gemm_edges[0].reshape(NCH, CH, 4).transpose(0, 2, 1)
    Wst = jnp.transpose(W[:, :, 0, :], (2, 0, 1))
    b2 = b.reshape(O, 1)
    G = _gather(xT, idx_arr)
    out2 = _conv(x2, G, Wst, b2)
    return out2[None, :, :, None]


# speedup vs baseline: 6.0252x; 1.0220x over previous
import jax
import jax.numpy as jnp
from jax import lax
from jax.experimental import pallas as pl
from jax.experimental.pallas import tpu as pltpu
from jax.experimental.pallas import tpu_sc as plsc

B, C, E, O, K = 1, 128, 160000, 128, 5

NC, NS = 2, 16
NW = NC * NS
CH = 128
NCH = E // CH

NSPLIT = 5
ES = E // NSPLIT
NCH_S = NCH // NSPLIT
ITERS = (NCH_S + NW - 1) // NW

BE = 1280
NBLK_S = ES // BE


def _sc_gather_body(xT_ref, idx_ref, g_ref, idx_v, rows_v,
                    gs0, gs1, gs2, gs3, ws0, ws1, ws2, ws3, isem):
    gsem = (gs0, gs1, gs2, gs3)
    wsem = (ws0, ws1, ws2, ws3)
    cid = lax.axis_index("c")
    sid = lax.axis_index("s")
    wid = sid * NC + cid

    pltpu.async_copy(idx_ref.at[wid], idx_v.at[0], isem)

    def chunk_body(i, carry):
        c = wid + i * NW
        p = lax.rem(i, 2)

        @pl.when(c < NCH_S)
        def _():
            pltpu.make_async_copy(idx_ref.at[c], idx_v.at[p], isem).wait()

            @pl.when(c + NW < NCH_S)
            def _():
                pltpu.async_copy(idx_ref.at[c + NW], idx_v.at[1 - p], isem)

            for j in range(4):
                @pl.when(i >= 1)
                def _():
                    pltpu.make_async_copy(
                        rows_v.at[j], g_ref.at[j, pl.ds(0, CH), :], wsem[j]
                    ).wait()

                pltpu.async_copy(
                    xT_ref.at[idx_v.at[p, j]], rows_v.at[j], gsem[j]
                )
            for j in range(4):
                pltpu.make_async_copy(
                    xT_ref.at[idx_v.at[p, j]], rows_v.at[j], gsem[j]
                ).wait()
                pltpu.async_copy(
                    rows_v.at[j], g_ref.at[j, pl.ds(c * CH, CH), :], wsem[j]
                )

        return carry

    lax.fori_loop(0, ITERS, chunk_body, 0)
    for j in range(4):
        pltpu.make_async_copy(
            rows_v.at[j], g_ref.at[j, pl.ds(0, CH), :], wsem[j]
        ).wait()


_gather = pl.kernel(
    _sc_gather_body,
    out_type=jax.ShapeDtypeStruct((4, ES, C), jnp.float32),
    mesh=plsc.VectorSubcoreMesh(
        core_axis_name="c", subcore_axis_name="s", num_cores=NC, num_subcores=NS
    ),
    scratch_types=[
        pltpu.VMEM((2, 4, CH), jnp.int32),
        pltpu.VMEM((4, CH, C), jnp.float32),
    ] + [pltpu.SemaphoreType.DMA] * 9,
)


def _tc_conv_body(x_ref, g_ref, w_ref, b_ref, *rest):
    o_ref = rest[-1]
    r1 = g_ref[0]
    r2 = g_ref[1]
    r3 = g_ref[2]
    r4 = g_ref[3]
    dn = (((1,), (1,)), ((), ()))
    a = jnp.dot(w_ref[0], x_ref[...], preferred_element_type=jnp.float32)
    a += lax.dot_general(w_ref[1], r1 + r3, dn, preferred_element_type=jnp.float32)
    a += lax.dot_general(w_ref[2], r2 + r4, dn, preferred_element_type=jnp.float32)
    a += lax.dot_general(w_ref[3], jnp.abs(r1 - r3), dn, preferred_element_type=jnp.float32)
    a += lax.dot_general(w_ref[4], jnp.abs(r2 - r4), dn, preferred_element_type=jnp.float32)
    o_ref[...] = a + b_ref[...]


def _make_conv(k):
    off = k * NBLK_S
    in_specs = [
        pl.BlockSpec((C, BE), lambda i: (0, i + off)),
        pl.BlockSpec((4, BE, C), lambda i: (0, i, 0)),
        pl.BlockSpec((K, O, C), lambda i: (0, 0, 0)),
        pl.BlockSpec((O, 1), lambda i: (0, 0)),
    ]
    aliases = {}
    if k > 0:
        in_specs.append(pl.BlockSpec(memory_space=pl.ANY))
        aliases = {4: 0}
    return pl.pallas_call(
        _tc_conv_body,
        grid=(NBLK_S,),
        in_specs=in_specs,
        out_specs=pl.BlockSpec((O, BE), lambda i: (0, i + off)),
        out_shape=jax.ShapeDtypeStruct((O, E), jnp.float32),
        input_output_aliases=aliases,
    )


_convs = [_make_conv(k) for k in range(NSPLIT)]


def kernel(x, gemm_edges, W, b):
    x2 = x[0]
    xT = x2.T
    idx_arr = gemm_edges[0].reshape(NCH, CH, 4).transpose(0, 2, 1)
    Wst = jnp.transpose(W[:, :, 0, :], (2, 0, 1))
    b2 = b.reshape(O, 1)

    gs = [
        _gather(xT, idx_arr[k * NCH_S:(k + 1) * NCH_S])
        for k in range(NSPLIT)
    ]
    out2 = _convs[0](x2, gs[0], Wst, b2)
    for k in range(1, NSPLIT):
        out2 = _convs[k](x2, gs[k], Wst, b2, out2)
    return out2[None, :, :, None]


# speedup vs baseline: 6.0527x; 1.0046x over previous
import jax
import jax.numpy as jnp
from jax import lax
from jax.experimental import pallas as pl
from jax.experimental.pallas import tpu as pltpu
from jax.experimental.pallas import tpu_sc as plsc

B, C, E, O, K = 1, 128, 160000, 128, 5

NC, NS = 2, 16
NW = NC * NS
CH = 128
NCH = E // CH

NSPLIT = 5
ES = E // NSPLIT
NCH_S = NCH // NSPLIT
ITERS = (NCH_S + NW - 1) // NW

BE = 3200
NBLK_S = ES // BE


def _sc_gather_body(xT_ref, idx_ref, g_ref, idx_v, rows_v,
                    gs0, gs1, gs2, gs3, ws0, ws1, ws2, ws3, isem):
    gsem = (gs0, gs1, gs2, gs3)
    wsem = (ws0, ws1, ws2, ws3)
    cid = lax.axis_index("c")
    sid = lax.axis_index("s")
    wid = sid * NC + cid

    pltpu.async_copy(idx_ref.at[wid], idx_v.at[0], isem)

    def chunk_body(i, carry):
        c = wid + i * NW
        p = lax.rem(i, 2)

        @pl.when(c < NCH_S)
        def _():
            pltpu.make_async_copy(idx_ref.at[c], idx_v.at[p], isem).wait()

            @pl.when(c + NW < NCH_S)
            def _():
                pltpu.async_copy(idx_ref.at[c + NW], idx_v.at[1 - p], isem)

            for j in range(4):
                @pl.when(i >= 1)
                def _():
                    pltpu.make_async_copy(
                        rows_v.at[j], g_ref.at[j, pl.ds(0, CH), :], wsem[j]
                    ).wait()

                pltpu.async_copy(
                    xT_ref.at[idx_v.at[p, j]], rows_v.at[j], gsem[j]
                )
            for j in range(4):
                pltpu.make_async_copy(
                    xT_ref.at[idx_v.at[p, j]], rows_v.at[j], gsem[j]
                ).wait()
                pltpu.async_copy(
                    rows_v.at[j], g_ref.at[j, pl.ds(c * CH, CH), :], wsem[j]
                )

        return carry

    lax.fori_loop(0, ITERS, chunk_body, 0)
    for j in range(4):
        pltpu.make_async_copy(
            rows_v.at[j], g_ref.at[j, pl.ds(0, CH), :], wsem[j]
        ).wait()


_gather = pl.kernel(
    _sc_gather_body,
    out_type=jax.ShapeDtypeStruct((4, ES, C), jnp.float32),
    mesh=plsc.VectorSubcoreMesh(
        core_axis_name="c", subcore_axis_name="s", num_cores=NC, num_subcores=NS
    ),
    scratch_types=[
        pltpu.VMEM((2, 4, CH), jnp.int32),
        pltpu.VMEM((4, CH, C), jnp.float32),
    ] + [pltpu.SemaphoreType.DMA] * 9,
)


def _tc_conv_body(x_ref, g_ref, w_ref, b_ref, *rest):
    o_ref = rest[-1]
    r1 = g_ref[0]
    r2 = g_ref[1]
    r3 = g_ref[2]
    r4 = g_ref[3]
    dn = (((1,), (1,)), ((), ()))
    a = jnp.dot(w_ref[0], x_ref[...], preferred_element_type=jnp.float32)
    a += lax.dot_general(w_ref[1], r1 + r3, dn, preferred_element_type=jnp.float32)
    a += lax.dot_general(w_ref[2], r2 + r4, dn, preferred_element_type=jnp.float32)
    a += lax.dot_general(w_ref[3], jnp.abs(r1 - r3), dn, preferred_element_type=jnp.float32)
    a += lax.dot_general(w_ref[4], jnp.abs(r2 - r4), dn, preferred_element_type=jnp.float32)
    o_ref[...] = a + b_ref[...]


def _make_conv(k):
    off = k * NBLK_S
    in_specs = [
        pl.BlockSpec((C, BE), lambda i: (0, i + off)),
        pl.BlockSpec((4, BE, C), lambda i: (0, i, 0)),
        pl.BlockSpec((K, O, C), lambda i: (0, 0, 0)),
        pl.BlockSpec((O, 1), lambda i: (0, 0)),
    ]
    aliases = {}
    if k > 0:
        in_specs.append(pl.BlockSpec(memory_space=pl.ANY))
        aliases = {4: 0}
    return pl.pallas_call(
        _tc_conv_body,
        grid=(NBLK_S,),
        in_specs=in_specs,
        out_specs=pl.BlockSpec((O, BE), lambda i: (0, i + off)),
        out_shape=jax.ShapeDtypeStruct((O, E), jnp.float32),
        input_output_aliases=aliases,
    )


_convs = [_make_conv(k) for k in range(NSPLIT)]


def kernel(x, gemm_edges, W, b):
    x2 = x[0]
    xT = x2.T
    idx_arr = gemm_edges[0].reshape(NCH, CH, 4).transpose(0, 2, 1)
    Wst = jnp.transpose(W[:, :, 0, :], (2, 0, 1))
    b2 = b.reshape(O, 1)

    gs = [
        _gather(xT, idx_arr[k * NCH_S:(k + 1) * NCH_S])
        for k in range(NSPLIT)
    ]
    out2 = _convs[0](x2, gs[0], Wst, b2)
    for k in range(1, NSPLIT):
        out2 = _convs[k](x2, gs[k], Wst, b2, out2)
    return out2[None, :, :, None]
